# baseline (speedup 1.0000x reference)
import functools

import jax
import jax.numpy as jnp
from jax import lax
from jax.experimental import pallas as pl
from jax.experimental.pallas import tpu as pltpu
import jax.experimental.pallas.tpu_sc as plsc

N = 2048
D = 768
E = 8
F = 3072
K = 2
P2 = 2 * N
BLK = 128
NB = P2 // BLK + E
P = NB * BLK

NC = 2
NS = 16
NW = NC * NS
LANES = 16

@functools.cache
def _sc_mesh():
    return plsc.VectorSubcoreMesh(
        core_axis_name="c", subcore_axis_name="s",
        num_cores=NC, num_subcores=NS)


def _router_body(rwT_ref, rb_ref, xT_ref, wpair_ref, slot_ref, be_ref, aux_ref):
    logits = jnp.dot(rwT_ref[...], xT_ref[...],
                     preferred_element_type=jnp.float32) + rb_ref[...]
    m = jnp.max(logits, axis=0, keepdims=True)
    ex = jnp.exp(logits - m)
    probs = ex / jnp.sum(ex, axis=0, keepdims=True)

    mp = jnp.sum(probs, axis=1, keepdims=True) * (1.0 / N)
    mu = jnp.sum(mp) * (1.0 / E)
    aux_ref[0, 0] = jnp.sum((mp - mu) ** 2) * (1.0 / (E - 1))

    eid = lax.broadcasted_iota(jnp.int32, (E, N), 0)
    p1 = jnp.max(probs, axis=0, keepdims=True)
    i1 = jnp.min(jnp.where(probs == p1, eid, E), axis=0, keepdims=True)
    oh1 = eid == i1
    masked = jnp.where(oh1, -1.0, probs)
    p2 = jnp.max(masked, axis=0, keepdims=True)
    i2 = jnp.min(jnp.where(masked == p2, eid, E), axis=0, keepdims=True)
    oh2 = eid == i2
    sw = p1 + p2
    wpair_ref[...] = jnp.concatenate([p1 / sw, p2 / sw], axis=1)

    oh = jnp.concatenate([oh1, oh2], axis=1).astype(jnp.float32)
    c = oh
    sh = 1
    while sh < P2:
        c = c + jnp.concatenate(
            [jnp.zeros((E, sh), jnp.float32), c[:, : P2 - sh]], axis=1)
        sh *= 2
    counts = c[:, P2 - 1 : P2]
    rank = c - oh
    caps = jnp.ceil(counts * (1.0 / BLK)) * BLK
    ic = caps
    sh = 1
    while sh < E:
        ic = ic + jnp.concatenate(
            [jnp.zeros((sh, 1), jnp.float32), ic[: E - sh]], axis=0)
        sh *= 2
    gs = ic - caps
    slot_f = jnp.sum(oh * (gs + rank), axis=0, keepdims=True)
    slot_ref[...] = slot_f.astype(jnp.int32)

    bstart = lax.broadcasted_iota(jnp.int32, (E, NB), 1) * BLK
    be = jnp.sum((bstart >= ic.astype(jnp.int32)).astype(jnp.int32),
                 axis=0, keepdims=True)
    be_ref[...] = jnp.minimum(be, E - 1)


_router = pl.pallas_call(
    _router_body,
    out_shape=(
        jax.ShapeDtypeStruct((1, P2), jnp.float32),
        jax.ShapeDtypeStruct((1, P2), jnp.int32),
        jax.ShapeDtypeStruct((1, NB), jnp.int32),
        jax.ShapeDtypeStruct((1, 1), jnp.float32),
    ),
    out_specs=(
        pl.BlockSpec(memory_space=pltpu.VMEM),
        pl.BlockSpec(memory_space=pltpu.VMEM),
        pl.BlockSpec(memory_space=pltpu.VMEM),
        pl.BlockSpec(memory_space=pltpu.SMEM),
    ),
)


@functools.cache
def _build_kernel():
    return pl.kernel(
        _build_body,
        out_type=(
            jax.ShapeDtypeStruct((P,), jnp.int32),
            jax.ShapeDtypeStruct((P,), jnp.float32),
        ),
        mesh=_sc_mesh(),
        compiler_params=pltpu.CompilerParams(needs_layout_passes=False),
        scratch_types=(
            pltpu.VMEM((P,), jnp.int32),
            pltpu.VMEM((P,), jnp.float32),
            pltpu.VMEM((P2,), jnp.int32),
            pltpu.VMEM((P2,), jnp.int32),
            pltpu.VMEM((P2,), jnp.float32),
        ),
    )


def _build_body(slot_hbm, tok_hbm, wp_hbm, gidx_hbm, wslot_hbm,
                g_v, w_v, slot_v, tok_v, wp_v):
    cid = lax.axis_index("c")
    sid = lax.axis_index("s")

    @pl.when(jnp.logical_and(cid == 0, sid == 0))
    def _():
        pltpu.sync_copy(slot_hbm, slot_v)
        pltpu.sync_copy(tok_hbm, tok_v)
        pltpu.sync_copy(wp_hbm, wp_v)

        def _zero(i, carry):
            g_v[pl.ds(i * LANES, LANES)] = jnp.zeros((LANES,), jnp.int32)
            w_v[pl.ds(i * LANES, LANES)] = jnp.zeros((LANES,), jnp.float32)
            return carry
        lax.fori_loop(0, P // LANES, _zero, 0)

        def _scat(i, carry):
            sl = slot_v[pl.ds(i * LANES, LANES)]
            plsc.store_scatter(g_v, [sl], tok_v[pl.ds(i * LANES, LANES)])
            plsc.store_scatter(w_v, [sl], wp_v[pl.ds(i * LANES, LANES)])
            return carry
        lax.fori_loop(0, P2 // LANES, _scat, 0)

        pltpu.sync_copy(g_v, gidx_hbm)
        pltpu.sync_copy(w_v, wslot_hbm)


_ROWS_W = P // NW
_CHUNK = _ROWS_W // 2


_D2 = D // 2


@functools.cache
def _dispatch_kernel():
    return pl.kernel(
        _dispatch_body,
        out_type=jax.ShapeDtypeStruct((P, _D2), jnp.int32),
        mesh=_sc_mesh(),
        compiler_params=pltpu.CompilerParams(needs_layout_passes=False),
        scratch_types=(
            pltpu.VMEM((_CHUNK,), jnp.int32),
            pltpu.VMEM((_CHUNK, _D2), jnp.int32),
            pltpu.SemaphoreType.DMA,
        ),
    )


def _dispatch_body(x_hbm, gidx_hbm, xs_hbm, idx_v, rows_v, sem):
    cid = lax.axis_index("c")
    sid = lax.axis_index("s")
    wid = sid * NC + cid
    for ch in range(_ROWS_W // _CHUNK):
        base = wid * _ROWS_W + ch * _CHUNK
        pltpu.sync_copy(gidx_hbm.at[pl.ds(base, _CHUNK)], idx_v)
        pltpu.async_copy(x_hbm.at[idx_v], rows_v, sem).wait()
        pltpu.sync_copy(rows_v, xs_hbm.at[pl.ds(base, _CHUNK)])


def _ffn_body(be_ref, xs_ref, w1_ref, b1_ref, w2_ref, b2_ref, ws_ref, out_ref):
    h = jnp.dot(xs_ref[...], w1_ref[0],
                preferred_element_type=jnp.float32) + b1_ref[0]
    h = 0.5 * h * (1.0 + lax.erf(h * 0.7071067811865476))
    y = jnp.dot(h.astype(jnp.bfloat16), w2_ref[0],
                preferred_element_type=jnp.float32) + b2_ref[0]
    out_ref[...] = y * ws_ref[...]


_ffn = pl.pallas_call(
    _ffn_body,
    grid_spec=pltpu.PrefetchScalarGridSpec(
        num_scalar_prefetch=1,
        grid=(NB,),
        in_specs=[
            pl.BlockSpec((BLK, D), lambda i, be: (i, 0)),
            pl.BlockSpec((1, D, F), lambda i, be: (be[i], 0, 0)),
            pl.BlockSpec((1, 1, F), lambda i, be: (be[i], 0, 0)),
            pl.BlockSpec((1, F, D), lambda i, be: (be[i], 0, 0)),
            pl.BlockSpec((1, 1, D), lambda i, be: (be[i], 0, 0)),
            pl.BlockSpec((BLK, 1), lambda i, be: (i, 0)),
        ],
        out_specs=pl.BlockSpec((BLK, D), lambda i, be: (i, 0)),
    ),
    out_shape=jax.ShapeDtypeStruct((P, D), jnp.float32),
    compiler_params=pltpu.CompilerParams(
        dimension_semantics=("arbitrary",)),
)


_TOK_W = N // NW


@functools.cache
def _combine_kernel():
    return pl.kernel(
        _combine_body,
        out_type=jax.ShapeDtypeStruct((N, D), jnp.float32),
        mesh=_sc_mesh(),
        compiler_params=pltpu.CompilerParams(needs_layout_passes=False),
        scratch_types=(
            pltpu.VMEM((_TOK_W,), jnp.int32),
            pltpu.VMEM((_TOK_W,), jnp.int32),
            pltpu.VMEM((_TOK_W, D), jnp.float32),
            pltpu.VMEM((_TOK_W, D), jnp.float32),
            pltpu.SemaphoreType.DMA,
            pltpu.SemaphoreType.DMA,
        ),
    )


def _combine_body(ysw_hbm, s1_hbm, s2_hbm, out_hbm,
                  idx1_v, idx2_v, buf1_v, buf2_v, sem1, sem2):
    cid = lax.axis_index("c")
    sid = lax.axis_index("s")
    wid = sid * NC + cid
    base = wid * _TOK_W
    pltpu.sync_copy(s1_hbm.at[pl.ds(base, _TOK_W)], idx1_v)
    pltpu.sync_copy(s2_hbm.at[pl.ds(base, _TOK_W)], idx2_v)
    cp1 = pltpu.async_copy(ysw_hbm.at[idx1_v], buf1_v, sem1)
    cp2 = pltpu.async_copy(ysw_hbm.at[idx2_v], buf2_v, sem2)
    cp1.wait()
    cp2.wait()

    def _row(r, carry):
        for j in range(D // LANES):
            sl = pl.ds(j * LANES, LANES)
            buf1_v[r, sl] = buf1_v[r, sl] + buf2_v[r, sl]
        return carry
    lax.fori_loop(0, _TOK_W, _row, 0)
    pltpu.sync_copy(buf1_v, out_hbm.at[pl.ds(base, _TOK_W)])


def kernel(x, router_W, router_b, W1, b1, W2, b2):
    x_flat = x.reshape(N, D)
    wpair2d, slot2d, be2d, aux = _router(
        router_W.T, router_b.reshape(E, 1), x_flat.T)
    slot = slot2d.reshape(P2)
    tok = jnp.concatenate(
        [jnp.arange(N, dtype=jnp.int32), jnp.arange(N, dtype=jnp.int32)])
    gidx, wslot = _build_kernel()(slot, tok, wpair2d.reshape(P2))
    x_i32 = lax.bitcast_convert_type(
        x_flat.astype(jnp.bfloat16).reshape(N, _D2, 2), jnp.int32)
    xs_i32 = _dispatch_kernel()(x_i32, gidx)
    xs = lax.bitcast_convert_type(xs_i32, jnp.bfloat16).reshape(P, D)
    ysw = _ffn(be2d.reshape(NB), xs, W1.astype(jnp.bfloat16),
               b1.reshape(E, 1, F), W2.astype(jnp.bfloat16),
               b2.reshape(E, 1, D), wslot.reshape(P, 1))
    out = _combine_kernel()(ysw, slot[:N], slot[N:])
    return out.reshape(x.shape), aux[0, 0]


# speedup vs baseline: 1.6728x; 1.6728x over previous
import functools

import jax
import jax.numpy as jnp
from jax import lax
from jax.experimental import pallas as pl
from jax.experimental.pallas import tpu as pltpu
import jax.experimental.pallas.tpu_sc as plsc

N = 2048
D = 768
E = 8
F = 3072
K = 2
P2 = 2 * N
BLK = 128
NB = P2 // BLK + E
P = NB * BLK

NC = 2
NS = 16
NW = NC * NS
LANES = 16

@functools.cache
def _sc_mesh():
    return plsc.VectorSubcoreMesh(
        core_axis_name="c", subcore_axis_name="s",
        num_cores=NC, num_subcores=NS)


def _router_body(rwT_ref, rb_ref, xT_ref, wpair_ref, slot_ref, be_ref, aux_ref):
    logits = jnp.dot(rwT_ref[...], xT_ref[...],
                     preferred_element_type=jnp.float32) + rb_ref[...]
    m = jnp.max(logits, axis=0, keepdims=True)
    ex = jnp.exp(logits - m)
    probs = ex / jnp.sum(ex, axis=0, keepdims=True)

    mp = jnp.sum(probs, axis=1, keepdims=True) * (1.0 / N)
    mu = jnp.sum(mp) * (1.0 / E)
    aux_ref[0, 0] = jnp.sum((mp - mu) ** 2) * (1.0 / (E - 1))

    eid = lax.broadcasted_iota(jnp.int32, (E, N), 0)
    p1 = jnp.max(probs, axis=0, keepdims=True)
    i1 = jnp.min(jnp.where(probs == p1, eid, E), axis=0, keepdims=True)
    oh1 = eid == i1
    masked = jnp.where(oh1, -1.0, probs)
    p2 = jnp.max(masked, axis=0, keepdims=True)
    i2 = jnp.min(jnp.where(masked == p2, eid, E), axis=0, keepdims=True)
    oh2 = eid == i2
    sw = p1 + p2
    wpair_ref[...] = jnp.concatenate([p1 / sw, p2 / sw], axis=1)

    oh = jnp.concatenate([oh1, oh2], axis=1).astype(jnp.float32)
    c = oh
    sh = 1
    while sh < P2:
        c = c + jnp.concatenate(
            [jnp.zeros((E, sh), jnp.float32), c[:, : P2 - sh]], axis=1)
        sh *= 2
    counts = c[:, P2 - 1 : P2]
    rank = c - oh
    caps = jnp.ceil(counts * (1.0 / BLK)) * BLK
    ic = caps
    sh = 1
    while sh < E:
        ic = ic + jnp.concatenate(
            [jnp.zeros((sh, 1), jnp.float32), ic[: E - sh]], axis=0)
        sh *= 2
    gs = ic - caps
    slot_f = jnp.sum(oh * (gs + rank), axis=0, keepdims=True)
    slot_ref[...] = slot_f.astype(jnp.int32)

    bstart = lax.broadcasted_iota(jnp.int32, (E, NB), 1) * BLK
    be = jnp.sum((bstart >= ic.astype(jnp.int32)).astype(jnp.int32),
                 axis=0, keepdims=True)
    be_ref[...] = jnp.minimum(be, E - 1)


_router = pl.pallas_call(
    _router_body,
    out_shape=(
        jax.ShapeDtypeStruct((1, P2), jnp.float32),
        jax.ShapeDtypeStruct((1, P2), jnp.int32),
        jax.ShapeDtypeStruct((1, NB), jnp.int32),
        jax.ShapeDtypeStruct((1, 1), jnp.float32),
    ),
    out_specs=(
        pl.BlockSpec(memory_space=pltpu.VMEM),
        pl.BlockSpec(memory_space=pltpu.VMEM),
        pl.BlockSpec(memory_space=pltpu.VMEM),
        pl.BlockSpec(memory_space=pltpu.SMEM),
    ),
)


@functools.cache
def _build_kernel():
    return pl.kernel(
        _build_body,
        out_type=(
            jax.ShapeDtypeStruct((P,), jnp.int32),
            jax.ShapeDtypeStruct((P,), jnp.float32),
        ),
        mesh=_sc_mesh(),
        compiler_params=pltpu.CompilerParams(needs_layout_passes=False),
        scratch_types=(
            pltpu.VMEM((P,), jnp.int32),
            pltpu.VMEM((P,), jnp.float32),
            pltpu.VMEM((P2,), jnp.int32),
            pltpu.VMEM((P2,), jnp.int32),
            pltpu.VMEM((P2,), jnp.float32),
        ),
    )


def _build_body(slot_hbm, tok_hbm, wp_hbm, gidx_hbm, wslot_hbm,
                g_v, w_v, slot_v, tok_v, wp_v):
    cid = lax.axis_index("c")
    sid = lax.axis_index("s")

    @pl.when(jnp.logical_and(cid == 0, sid == 0))
    def _():
        pltpu.sync_copy(slot_hbm, slot_v)
        pltpu.sync_copy(tok_hbm, tok_v)
        pltpu.sync_copy(wp_hbm, wp_v)

        def _zero(i, carry):
            g_v[pl.ds(i * LANES, LANES)] = jnp.zeros((LANES,), jnp.int32)
            w_v[pl.ds(i * LANES, LANES)] = jnp.zeros((LANES,), jnp.float32)
            return carry
        lax.fori_loop(0, P // LANES, _zero, 0)

        def _scat(i, carry):
            sl = slot_v[pl.ds(i * LANES, LANES)]
            plsc.store_scatter(g_v, [sl], tok_v[pl.ds(i * LANES, LANES)])
            plsc.store_scatter(w_v, [sl], wp_v[pl.ds(i * LANES, LANES)])
            return carry
        lax.fori_loop(0, P2 // LANES, _scat, 0)

        pltpu.sync_copy(g_v, gidx_hbm)
        pltpu.sync_copy(w_v, wslot_hbm)


_ROWS_W = P // NW
_CHUNK = _ROWS_W // 2


_NCHUNK = 4
_CH = _ROWS_W // _NCHUNK


@functools.cache
def _dispatch_kernel():
    return pl.kernel(
        _dispatch_body,
        out_type=jax.ShapeDtypeStruct((P, D), jnp.float32),
        mesh=_sc_mesh(),
        compiler_params=pltpu.CompilerParams(needs_layout_passes=False),
        scratch_types=(
            pltpu.VMEM((_ROWS_W,), jnp.int32),
            pltpu.VMEM((_CH, D), jnp.float32),
            pltpu.VMEM((_CH, D), jnp.float32),
            pltpu.SemaphoreType.DMA,
            pltpu.SemaphoreType.DMA,
            pltpu.SemaphoreType.DMA,
            pltpu.SemaphoreType.DMA,
        ),
    )


def _dispatch_body(x_hbm, gidx_hbm, xs_hbm, idx_v, rows0_v, rows1_v,
                   gsem0, gsem1, wsem0, wsem1):
    cid = lax.axis_index("c")
    sid = lax.axis_index("s")
    wid = sid * NC + cid
    base = wid * _ROWS_W
    pltpu.sync_copy(gidx_hbm.at[pl.ds(base, _ROWS_W)], idx_v)
    bufs = (rows0_v, rows1_v)
    gsems = (gsem0, gsem1)
    wsems = (wsem0, wsem1)
    writes = [None, None]
    gathers = [None] * _NCHUNK
    for ch in range(2):
        gathers[ch] = pltpu.async_copy(
            x_hbm.at[idx_v.at[pl.ds(ch * _CH, _CH)]], bufs[ch], gsems[ch])
    for ch in range(_NCHUNK):
        b = ch % 2
        gathers[ch].wait()
        writes[b] = pltpu.async_copy(
            bufs[b], xs_hbm.at[pl.ds(base + ch * _CH, _CH)], wsems[b])
        nxt = ch + 2
        if nxt < _NCHUNK:
            writes[b].wait()
            writes[b] = None
            gathers[nxt] = pltpu.async_copy(
                x_hbm.at[idx_v.at[pl.ds(nxt * _CH, _CH)]], bufs[b], gsems[b])
    for w in writes:
        if w is not None:
            w.wait()


def _ffn_body(be_ref, xs_ref, w1_ref, b1_ref, w2_ref, b2_ref, ws_ref, out_ref):
    h = jnp.dot(xs_ref[...], w1_ref[0],
                preferred_element_type=jnp.float32) + b1_ref[0]
    h = 0.5 * h * (1.0 + lax.erf(h * 0.7071067811865476))
    y = jnp.dot(h, w2_ref[0],
                preferred_element_type=jnp.float32) + b2_ref[0]
    out_ref[...] = y * ws_ref[...]


_ffn = pl.pallas_call(
    _ffn_body,
    grid_spec=pltpu.PrefetchScalarGridSpec(
        num_scalar_prefetch=1,
        grid=(NB,),
        in_specs=[
            pl.BlockSpec((BLK, D), lambda i, be: (i, 0)),
            pl.BlockSpec((1, D, F), lambda i, be: (be[i], 0, 0)),
            pl.BlockSpec((1, 1, F), lambda i, be: (be[i], 0, 0)),
            pl.BlockSpec((1, F, D), lambda i, be: (be[i], 0, 0)),
            pl.BlockSpec((1, 1, D), lambda i, be: (be[i], 0, 0)),
            pl.BlockSpec((BLK, 1), lambda i, be: (i, 0)),
        ],
        out_specs=pl.BlockSpec((BLK, D), lambda i, be: (i, 0)),
    ),
    out_shape=jax.ShapeDtypeStruct((P, D), jnp.float32),
    compiler_params=pltpu.CompilerParams(
        dimension_semantics=("arbitrary",)),
)


_TOK_W = N // NW


@functools.cache
def _combine_kernel():
    return pl.kernel(
        _combine_body,
        out_type=jax.ShapeDtypeStruct((N, D), jnp.float32),
        mesh=_sc_mesh(),
        compiler_params=pltpu.CompilerParams(needs_layout_passes=False),
        scratch_types=(
            pltpu.VMEM((_TOK_W,), jnp.int32),
            pltpu.VMEM((_TOK_W,), jnp.int32),
            pltpu.VMEM((_TOK_W, D), jnp.float32),
            pltpu.VMEM((_TOK_W, D), jnp.float32),
            pltpu.SemaphoreType.DMA,
            pltpu.SemaphoreType.DMA,
        ),
    )


def _combine_body(ysw_hbm, s1_hbm, s2_hbm, out_hbm,
                  idx1_v, idx2_v, buf1_v, buf2_v, sem1, sem2):
    cid = lax.axis_index("c")
    sid = lax.axis_index("s")
    wid = sid * NC + cid
    base = wid * _TOK_W
    pltpu.sync_copy(s1_hbm.at[pl.ds(base, _TOK_W)], idx1_v)
    pltpu.sync_copy(s2_hbm.at[pl.ds(base, _TOK_W)], idx2_v)
    cp1 = pltpu.async_copy(ysw_hbm.at[idx1_v], buf1_v, sem1)
    cp2 = pltpu.async_copy(ysw_hbm.at[idx2_v], buf2_v, sem2)
    cp1.wait()
    cp2.wait()

    def _row(r, carry):
        for j in range(D // LANES):
            sl = pl.ds(j * LANES, LANES)
            buf1_v[r, sl] = buf1_v[r, sl] + buf2_v[r, sl]
        return carry
    lax.fori_loop(0, _TOK_W, _row, 0)
    pltpu.sync_copy(buf1_v, out_hbm.at[pl.ds(base, _TOK_W)])


def kernel(x, router_W, router_b, W1, b1, W2, b2):
    x_flat = x.reshape(N, D)
    wpair2d, slot2d, be2d, aux = _router(
        router_W.T, router_b.reshape(E, 1), x_flat.T)
    slot = slot2d.reshape(P2)
    tok = jnp.concatenate(
        [jnp.arange(N, dtype=jnp.int32), jnp.arange(N, dtype=jnp.int32)])
    gidx, wslot = _build_kernel()(slot, tok, wpair2d.reshape(P2))
    xs = _dispatch_kernel()(x_flat, gidx)
    ysw = _ffn(be2d.reshape(NB), xs, W1, b1.reshape(E, 1, F), W2,
               b2.reshape(E, 1, D), wslot.reshape(P, 1))
    out = _combine_kernel()(ysw, slot[:N], slot[N:])
    return out.reshape(x.shape), aux[0, 0]


# speedup vs baseline: 1.9322x; 1.1551x over previous
import functools

import jax
import jax.numpy as jnp
from jax import lax
from jax.experimental import pallas as pl
from jax.experimental.pallas import tpu as pltpu
import jax.experimental.pallas.tpu_sc as plsc

N = 2048
D = 768
E = 8
F = 3072
K = 2
P2 = 2 * N
BLK = 128
NB = P2 // BLK + E
P = NB * BLK

NC = 2
NS = 16
NW = NC * NS
LANES = 16

@functools.cache
def _sc_mesh():
    return plsc.VectorSubcoreMesh(
        core_axis_name="c", subcore_axis_name="s",
        num_cores=NC, num_subcores=NS)


def _router_body(rwT_ref, rb_ref, xT_ref, wpair_ref, slot_ref, be_ref, aux_ref):
    logits = jnp.dot(rwT_ref[...], xT_ref[...],
                     preferred_element_type=jnp.float32) + rb_ref[...]
    m = jnp.max(logits, axis=0, keepdims=True)
    ex = jnp.exp(logits - m)
    probs = ex / jnp.sum(ex, axis=0, keepdims=True)

    mp = jnp.sum(probs, axis=1, keepdims=True) * (1.0 / N)
    mu = jnp.sum(mp) * (1.0 / E)
    aux_ref[0, 0] = jnp.sum((mp - mu) ** 2) * (1.0 / (E - 1))

    eid = lax.broadcasted_iota(jnp.int32, (E, N), 0)
    p1 = jnp.max(probs, axis=0, keepdims=True)
    i1 = jnp.min(jnp.where(probs == p1, eid, E), axis=0, keepdims=True)
    oh1 = eid == i1
    masked = jnp.where(oh1, -1.0, probs)
    p2 = jnp.max(masked, axis=0, keepdims=True)
    i2 = jnp.min(jnp.where(masked == p2, eid, E), axis=0, keepdims=True)
    oh2 = eid == i2
    sw = p1 + p2
    wpair_ref[...] = jnp.concatenate([p1 / sw, p2 / sw], axis=1)

    oh = jnp.concatenate([oh1, oh2], axis=1).astype(jnp.float32)
    c = oh
    sh = 1
    while sh < P2:
        c = c + jnp.concatenate(
            [jnp.zeros((E, sh), jnp.float32), c[:, : P2 - sh]], axis=1)
        sh *= 2
    counts = c[:, P2 - 1 : P2]
    rank = c - oh
    caps = jnp.ceil(counts * (1.0 / BLK)) * BLK
    ic = caps
    sh = 1
    while sh < E:
        ic = ic + jnp.concatenate(
            [jnp.zeros((sh, 1), jnp.float32), ic[: E - sh]], axis=0)
        sh *= 2
    gs = ic - caps
    slot_f = jnp.sum(oh * (gs + rank), axis=0, keepdims=True)
    slot_ref[...] = slot_f.astype(jnp.int32)

    bstart = lax.broadcasted_iota(jnp.int32, (E, NB), 1) * BLK
    be = jnp.sum((bstart >= ic.astype(jnp.int32)).astype(jnp.int32),
                 axis=0, keepdims=True)
    be_ref[...] = jnp.minimum(be, E - 1)


_router = pl.pallas_call(
    _router_body,
    out_shape=(
        jax.ShapeDtypeStruct((1, P2), jnp.float32),
        jax.ShapeDtypeStruct((1, P2), jnp.int32),
        jax.ShapeDtypeStruct((1, NB), jnp.int32),
        jax.ShapeDtypeStruct((1, 1), jnp.float32),
    ),
    out_specs=(
        pl.BlockSpec(memory_space=pltpu.VMEM),
        pl.BlockSpec(memory_space=pltpu.VMEM),
        pl.BlockSpec(memory_space=pltpu.VMEM),
        pl.BlockSpec(memory_space=pltpu.SMEM),
    ),
)


@functools.cache
def _build_kernel():
    return pl.kernel(
        _build_body,
        out_type=(
            jax.ShapeDtypeStruct((P,), jnp.int32),
            jax.ShapeDtypeStruct((P,), jnp.float32),
        ),
        mesh=_sc_mesh(),
        compiler_params=pltpu.CompilerParams(needs_layout_passes=False),
        scratch_types=(
            pltpu.VMEM((P,), jnp.int32),
            pltpu.VMEM((P,), jnp.float32),
            pltpu.VMEM((P2,), jnp.int32),
            pltpu.VMEM((P2,), jnp.int32),
            pltpu.VMEM((P2,), jnp.float32),
        ),
    )


def _build_body(slot_hbm, tok_hbm, wp_hbm, gidx_hbm, wslot_hbm,
                g_v, w_v, slot_v, tok_v, wp_v):
    cid = lax.axis_index("c")
    sid = lax.axis_index("s")

    @pl.when(jnp.logical_and(cid == 0, sid == 0))
    def _():
        pltpu.sync_copy(slot_hbm, slot_v)
        pltpu.sync_copy(tok_hbm, tok_v)
        pltpu.sync_copy(wp_hbm, wp_v)

        def _zero(i, carry):
            g_v[pl.ds(i * LANES, LANES)] = jnp.zeros((LANES,), jnp.int32)
            w_v[pl.ds(i * LANES, LANES)] = jnp.zeros((LANES,), jnp.float32)
            return carry
        lax.fori_loop(0, P // LANES, _zero, 0)

        def _scat(i, carry):
            sl = slot_v[pl.ds(i * LANES, LANES)]
            plsc.store_scatter(g_v, [sl], tok_v[pl.ds(i * LANES, LANES)])
            plsc.store_scatter(w_v, [sl], wp_v[pl.ds(i * LANES, LANES)])
            return carry
        lax.fori_loop(0, P2 // LANES, _scat, 0)

        pltpu.sync_copy(g_v, gidx_hbm)
        pltpu.sync_copy(w_v, wslot_hbm)


def _gather_block(x_ref, gidx_ref, buf, sem, blk):
    for r in range(BLK):
        tid = gidx_ref[blk * BLK + r]
        pltpu.make_async_copy(
            x_ref.at[pl.ds(tid, 1)], buf.at[pl.ds(r, 1)], sem).start()


def _ffn_body(be_ref, gidx_ref, x_ref, w1_ref, b1_ref, w2_ref, b2_ref,
              ws_ref, out_ref, xb0, xb1, sem0, sem1):
    i = pl.program_id(0)
    bufs = (xb0, xb1)
    sems = (sem0, sem1)

    @pl.when(i == 0)
    def _():
        _gather_block(x_ref, gidx_ref, xb0, sem0, 0)

    @pl.when(jnp.logical_and(i + 1 < NB, (i + 1) % 2 == 0))
    def _():
        _gather_block(x_ref, gidx_ref, xb0, sem0, i + 1)

    @pl.when(jnp.logical_and(i + 1 < NB, (i + 1) % 2 == 1))
    def _():
        _gather_block(x_ref, gidx_ref, xb1, sem1, i + 1)

    def _compute(buf, sem):
        pltpu.make_async_copy(x_ref.at[pl.ds(0, BLK)], buf, sem).wait()
        h = jnp.dot(buf[...], w1_ref[0],
                    preferred_element_type=jnp.float32) + b1_ref[0]
        h = 0.5 * h * (1.0 + lax.erf(h * 0.7071067811865476))
        y = jnp.dot(h, w2_ref[0],
                    preferred_element_type=jnp.float32) + b2_ref[0]
        out_ref[...] = y * ws_ref[...]

    @pl.when(i % 2 == 0)
    def _():
        _compute(xb0, sem0)

    @pl.when(i % 2 == 1)
    def _():
        _compute(xb1, sem1)


_ffn = pl.pallas_call(
    _ffn_body,
    grid_spec=pltpu.PrefetchScalarGridSpec(
        num_scalar_prefetch=2,
        grid=(NB,),
        in_specs=[
            pl.BlockSpec(memory_space=pl.ANY),
            pl.BlockSpec((1, D, F), lambda i, be, gidx: (be[i], 0, 0)),
            pl.BlockSpec((1, 1, F), lambda i, be, gidx: (be[i], 0, 0)),
            pl.BlockSpec((1, F, D), lambda i, be, gidx: (be[i], 0, 0)),
            pl.BlockSpec((1, 1, D), lambda i, be, gidx: (be[i], 0, 0)),
            pl.BlockSpec((BLK, 1), lambda i, be, gidx: (i, 0)),
        ],
        out_specs=pl.BlockSpec((BLK, D), lambda i, be, gidx: (i, 0)),
        scratch_shapes=[
            pltpu.VMEM((BLK, D), jnp.float32),
            pltpu.VMEM((BLK, D), jnp.float32),
            pltpu.SemaphoreType.DMA,
            pltpu.SemaphoreType.DMA,
        ],
    ),
    out_shape=jax.ShapeDtypeStruct((P, D), jnp.float32),
    compiler_params=pltpu.CompilerParams(
        dimension_semantics=("arbitrary",)),
)


_TOK_W = N // NW


@functools.cache
def _combine_kernel():
    return pl.kernel(
        _combine_body,
        out_type=jax.ShapeDtypeStruct((N, D), jnp.float32),
        mesh=_sc_mesh(),
        compiler_params=pltpu.CompilerParams(needs_layout_passes=False),
        scratch_types=(
            pltpu.VMEM((_TOK_W,), jnp.int32),
            pltpu.VMEM((_TOK_W,), jnp.int32),
            pltpu.VMEM((_TOK_W, D), jnp.float32),
            pltpu.VMEM((_TOK_W, D), jnp.float32),
            pltpu.SemaphoreType.DMA,
            pltpu.SemaphoreType.DMA,
        ),
    )


def _combine_body(ysw_hbm, s1_hbm, s2_hbm, out_hbm,
                  idx1_v, idx2_v, buf1_v, buf2_v, sem1, sem2):
    cid = lax.axis_index("c")
    sid = lax.axis_index("s")
    wid = sid * NC + cid
    base = wid * _TOK_W
    pltpu.sync_copy(s1_hbm.at[pl.ds(base, _TOK_W)], idx1_v)
    pltpu.sync_copy(s2_hbm.at[pl.ds(base, _TOK_W)], idx2_v)
    cp1 = pltpu.async_copy(ysw_hbm.at[idx1_v], buf1_v, sem1)
    cp2 = pltpu.async_copy(ysw_hbm.at[idx2_v], buf2_v, sem2)
    cp1.wait()
    cp2.wait()

    def _row(r, carry):
        for j in range(D // LANES):
            sl = pl.ds(j * LANES, LANES)
            buf1_v[r, sl] = buf1_v[r, sl] + buf2_v[r, sl]
        return carry
    lax.fori_loop(0, _TOK_W, _row, 0)
    pltpu.sync_copy(buf1_v, out_hbm.at[pl.ds(base, _TOK_W)])


def kernel(x, router_W, router_b, W1, b1, W2, b2):
    x_flat = x.reshape(N, D)
    wpair2d, slot2d, be2d, aux = _router(
        router_W.T, router_b.reshape(E, 1), x_flat.T)
    slot = slot2d.reshape(P2)
    tok = jnp.concatenate(
        [jnp.arange(N, dtype=jnp.int32), jnp.arange(N, dtype=jnp.int32)])
    gidx, wslot = _build_kernel()(slot, tok, wpair2d.reshape(P2))
    ysw = _ffn(be2d.reshape(NB), gidx, x_flat, W1, b1.reshape(E, 1, F), W2,
               b2.reshape(E, 1, D), wslot.reshape(P, 1))
    out = _combine_kernel()(ysw, slot[:N], slot[N:])
    return out.reshape(x.shape), aux[0, 0]


# speedup vs baseline: 2.0218x; 1.0464x over previous
import functools

import jax
import jax.numpy as jnp
from jax import lax
from jax.experimental import pallas as pl
from jax.experimental.pallas import tpu as pltpu
import jax.experimental.pallas.tpu_sc as plsc

N = 2048
D = 768
E = 8
F = 3072
K = 2
P2 = 2 * N
BLK = 128
NB = P2 // BLK + E
P = NB * BLK

NC = 2
NS = 16
NW = NC * NS
LANES = 16

@functools.cache
def _sc_mesh():
    return plsc.VectorSubcoreMesh(
        core_axis_name="c", subcore_axis_name="s",
        num_cores=NC, num_subcores=NS)


def _router_body(rwT_ref, rb_ref, xT_ref, wpair_ref, slot_ref, be_ref, aux_ref):
    logits = jnp.dot(rwT_ref[...], xT_ref[...],
                     preferred_element_type=jnp.float32) + rb_ref[...]
    m = jnp.max(logits, axis=0, keepdims=True)
    ex = jnp.exp(logits - m)
    probs = ex / jnp.sum(ex, axis=0, keepdims=True)

    mp = jnp.sum(probs, axis=1, keepdims=True) * (1.0 / N)
    mu = jnp.sum(mp) * (1.0 / E)
    aux_ref[0, 0] = jnp.sum((mp - mu) ** 2) * (1.0 / (E - 1))

    eid = lax.broadcasted_iota(jnp.int32, (E, N), 0)
    p1 = jnp.max(probs, axis=0, keepdims=True)
    i1 = jnp.min(jnp.where(probs == p1, eid, E), axis=0, keepdims=True)
    oh1 = eid == i1
    masked = jnp.where(oh1, -1.0, probs)
    p2 = jnp.max(masked, axis=0, keepdims=True)
    i2 = jnp.min(jnp.where(masked == p2, eid, E), axis=0, keepdims=True)
    oh2 = eid == i2
    sw = p1 + p2
    wpair_ref[...] = jnp.concatenate([p1 / sw, p2 / sw], axis=1)

    oh = jnp.concatenate([oh1, oh2], axis=1).astype(jnp.float32)
    c = oh
    sh = 1
    while sh < P2:
        c = c + jnp.concatenate(
            [jnp.zeros((E, sh), jnp.float32), c[:, : P2 - sh]], axis=1)
        sh *= 2
    counts = c[:, P2 - 1 : P2]
    rank = c - oh
    caps = jnp.ceil(counts * (1.0 / BLK)) * BLK
    ic = caps
    sh = 1
    while sh < E:
        ic = ic + jnp.concatenate(
            [jnp.zeros((sh, 1), jnp.float32), ic[: E - sh]], axis=0)
        sh *= 2
    gs = ic - caps
    slot_f = jnp.sum(oh * (gs + rank), axis=0, keepdims=True)
    slot_ref[...] = slot_f.astype(jnp.int32)

    bstart = lax.broadcasted_iota(jnp.int32, (E, NB), 1) * BLK
    be = jnp.sum((bstart >= ic.astype(jnp.int32)).astype(jnp.int32),
                 axis=0, keepdims=True)
    be = jnp.minimum(be, E - 1)
    be_prev = jnp.concatenate([be[:, :1], be[:, : NB - 1]], axis=1)
    gord = (be != be_prev).astype(jnp.int32)
    sh = 1
    while sh < NB:
        gord = gord + jnp.concatenate(
            [jnp.zeros((1, sh), jnp.int32), gord[:, : NB - sh]], axis=1)
        sh *= 2
    eidN = lax.broadcasted_iota(jnp.int32, (E, NB), 0)
    present = caps > 0.0
    cand = jnp.where((eidN > be) & present, eidN, E)
    nxe = jnp.min(cand, axis=0, keepdims=True)
    nxe = jnp.where(nxe == E, be, nxe)
    be_ref[...] = jnp.concatenate([be, gord, nxe], axis=1)


_router = pl.pallas_call(
    _router_body,
    out_shape=(
        jax.ShapeDtypeStruct((1, P2), jnp.float32),
        jax.ShapeDtypeStruct((1, P2), jnp.int32),
        jax.ShapeDtypeStruct((1, 3 * NB), jnp.int32),
        jax.ShapeDtypeStruct((1, 1), jnp.float32),
    ),
    out_specs=(
        pl.BlockSpec(memory_space=pltpu.VMEM),
        pl.BlockSpec(memory_space=pltpu.VMEM),
        pl.BlockSpec(memory_space=pltpu.VMEM),
        pl.BlockSpec(memory_space=pltpu.SMEM),
    ),
)


@functools.cache
def _build_kernel():
    return pl.kernel(
        _build_body,
        out_type=(
            jax.ShapeDtypeStruct((P,), jnp.int32),
            jax.ShapeDtypeStruct((P,), jnp.float32),
        ),
        mesh=_sc_mesh(),
        compiler_params=pltpu.CompilerParams(needs_layout_passes=False),
        scratch_types=(
            pltpu.VMEM((P,), jnp.int32),
            pltpu.VMEM((P,), jnp.float32),
            pltpu.VMEM((P2,), jnp.int32),
            pltpu.VMEM((P2,), jnp.int32),
            pltpu.VMEM((P2,), jnp.float32),
        ),
    )


def _build_body(slot_hbm, tok_hbm, wp_hbm, gidx_hbm, wslot_hbm,
                g_v, w_v, slot_v, tok_v, wp_v):
    cid = lax.axis_index("c")
    sid = lax.axis_index("s")

    @pl.when(jnp.logical_and(cid == 0, sid == 0))
    def _():
        pltpu.sync_copy(slot_hbm, slot_v)
        pltpu.sync_copy(tok_hbm, tok_v)
        pltpu.sync_copy(wp_hbm, wp_v)

        def _zero(i, carry):
            g_v[pl.ds(i * LANES, LANES)] = jnp.zeros((LANES,), jnp.int32)
            w_v[pl.ds(i * LANES, LANES)] = jnp.zeros((LANES,), jnp.float32)
            return carry
        lax.fori_loop(0, P // LANES, _zero, 0)

        def _scat(i, carry):
            sl = slot_v[pl.ds(i * LANES, LANES)]
            plsc.store_scatter(g_v, [sl], tok_v[pl.ds(i * LANES, LANES)])
            plsc.store_scatter(w_v, [sl], wp_v[pl.ds(i * LANES, LANES)])
            return carry
        lax.fori_loop(0, P2 // LANES, _scat, 0)

        pltpu.sync_copy(g_v, gidx_hbm)
        pltpu.sync_copy(w_v, wslot_hbm)


def _gather_block(x_ref, gidx_ref, buf, sem, blk):
    for r in range(BLK):
        tid = gidx_ref[blk * BLK + r]
        pltpu.make_async_copy(
            x_ref.at[pl.ds(tid, 1)], buf.at[pl.ds(r, 1)], sem).start()


def _ffn_body(arr_ref, gidx_ref, x_ref, w1_any, w2_any, b1_ref, b2_ref,
              ws_ref, out_ref, xb0, xb1, w1a, w1b, w2a, w2b,
              sem0, sem1, wsa, wsb):
    i = pl.program_id(0)
    e = arr_ref[i]
    g = arr_ref[NB + i]
    nxe = arr_ref[2 * NB + i]
    g_prev = arr_ref[NB + jnp.maximum(i - 1, 0)]
    first = jnp.logical_or(i == 0, g != g_prev)
    podd = g % 2 == 1

    @pl.when(i == 0)
    def _():
        _gather_block(x_ref, gidx_ref, xb0, sem0, 0)
        pltpu.make_async_copy(w1_any.at[e], w1a, wsa).start()
        pltpu.make_async_copy(w2_any.at[e], w2a, wsa).start()

    @pl.when(jnp.logical_and(i + 1 < NB, (i + 1) % 2 == 0))
    def _():
        _gather_block(x_ref, gidx_ref, xb0, sem0, i + 1)

    @pl.when(jnp.logical_and(i + 1 < NB, (i + 1) % 2 == 1))
    def _():
        _gather_block(x_ref, gidx_ref, xb1, sem1, i + 1)

    has_next = jnp.logical_and(first, nxe != e)

    @pl.when(jnp.logical_and(has_next, jnp.logical_not(podd)))
    def _():
        pltpu.make_async_copy(w1_any.at[nxe], w1b, wsb).start()
        pltpu.make_async_copy(w2_any.at[nxe], w2b, wsb).start()

    @pl.when(jnp.logical_and(has_next, podd))
    def _():
        pltpu.make_async_copy(w1_any.at[nxe], w1a, wsa).start()
        pltpu.make_async_copy(w2_any.at[nxe], w2a, wsa).start()

    @pl.when(jnp.logical_and(first, jnp.logical_not(podd)))
    def _():
        pltpu.make_async_copy(w1_any.at[e], w1a, wsa).wait()
        pltpu.make_async_copy(w2_any.at[e], w2a, wsa).wait()

    @pl.when(jnp.logical_and(first, podd))
    def _():
        pltpu.make_async_copy(w1_any.at[e], w1b, wsb).wait()
        pltpu.make_async_copy(w2_any.at[e], w2b, wsb).wait()

    def _compute(buf, sem, w1buf, w2buf):
        pltpu.make_async_copy(x_ref.at[pl.ds(0, BLK)], buf, sem).wait()
        h = jnp.dot(buf[...], w1buf[...],
                    preferred_element_type=jnp.float32) + b1_ref[0]
        h = 0.5 * h * (1.0 + lax.erf(h * 0.7071067811865476))
        y = jnp.dot(h, w2buf[...],
                    preferred_element_type=jnp.float32) + b2_ref[0]
        out_ref[...] = y * ws_ref[...]

    beven = i % 2 == 0

    @pl.when(jnp.logical_and(beven, jnp.logical_not(podd)))
    def _():
        _compute(xb0, sem0, w1a, w2a)

    @pl.when(jnp.logical_and(beven, podd))
    def _():
        _compute(xb0, sem0, w1b, w2b)

    @pl.when(jnp.logical_and(jnp.logical_not(beven), jnp.logical_not(podd)))
    def _():
        _compute(xb1, sem1, w1a, w2a)

    @pl.when(jnp.logical_and(jnp.logical_not(beven), podd))
    def _():
        _compute(xb1, sem1, w1b, w2b)


_ffn = pl.pallas_call(
    _ffn_body,
    grid_spec=pltpu.PrefetchScalarGridSpec(
        num_scalar_prefetch=2,
        grid=(NB,),
        in_specs=[
            pl.BlockSpec(memory_space=pl.ANY),
            pl.BlockSpec(memory_space=pl.ANY),
            pl.BlockSpec(memory_space=pl.ANY),
            pl.BlockSpec((1, 1, F), lambda i, arr, gidx: (arr[i], 0, 0)),
            pl.BlockSpec((1, 1, D), lambda i, arr, gidx: (arr[i], 0, 0)),
            pl.BlockSpec((BLK, 1), lambda i, arr, gidx: (i, 0)),
        ],
        out_specs=pl.BlockSpec((BLK, D), lambda i, arr, gidx: (i, 0)),
        scratch_shapes=[
            pltpu.VMEM((BLK, D), jnp.float32),
            pltpu.VMEM((BLK, D), jnp.float32),
            pltpu.VMEM((D, F), jnp.float32),
            pltpu.VMEM((D, F), jnp.float32),
            pltpu.VMEM((F, D), jnp.float32),
            pltpu.VMEM((F, D), jnp.float32),
            pltpu.SemaphoreType.DMA,
            pltpu.SemaphoreType.DMA,
            pltpu.SemaphoreType.DMA,
            pltpu.SemaphoreType.DMA,
        ],
    ),
    out_shape=jax.ShapeDtypeStruct((P, D), jnp.float32),
    compiler_params=pltpu.CompilerParams(
        dimension_semantics=("arbitrary",)),
)


_TOK_W = N // NW


@functools.cache
def _combine_kernel():
    return pl.kernel(
        _combine_body,
        out_type=jax.ShapeDtypeStruct((N, D), jnp.float32),
        mesh=_sc_mesh(),
        compiler_params=pltpu.CompilerParams(needs_layout_passes=False),
        scratch_types=(
            pltpu.VMEM((_TOK_W,), jnp.int32),
            pltpu.VMEM((_TOK_W,), jnp.int32),
            pltpu.VMEM((_TOK_W, D), jnp.float32),
            pltpu.VMEM((_TOK_W, D), jnp.float32),
            pltpu.SemaphoreType.DMA,
            pltpu.SemaphoreType.DMA,
        ),
    )


def _combine_body(ysw_hbm, s1_hbm, s2_hbm, out_hbm,
                  idx1_v, idx2_v, buf1_v, buf2_v, sem1, sem2):
    cid = lax.axis_index("c")
    sid = lax.axis_index("s")
    wid = sid * NC + cid
    base = wid * _TOK_W
    pltpu.sync_copy(s1_hbm.at[pl.ds(base, _TOK_W)], idx1_v)
    pltpu.sync_copy(s2_hbm.at[pl.ds(base, _TOK_W)], idx2_v)
    cp1 = pltpu.async_copy(ysw_hbm.at[idx1_v], buf1_v, sem1)
    cp2 = pltpu.async_copy(ysw_hbm.at[idx2_v], buf2_v, sem2)
    cp1.wait()
    cp2.wait()

    def _row(r, carry):
        for j in range(D // LANES):
            sl = pl.ds(j * LANES, LANES)
            buf1_v[r, sl] = buf1_v[r, sl] + buf2_v[r, sl]
        return carry
    lax.fori_loop(0, _TOK_W, _row, 0)
    pltpu.sync_copy(buf1_v, out_hbm.at[pl.ds(base, _TOK_W)])


def kernel(x, router_W, router_b, W1, b1, W2, b2):
    x_flat = x.reshape(N, D)
    wpair2d, slot2d, be2d, aux = _router(
        router_W.T, router_b.reshape(E, 1), x_flat.T)
    slot = slot2d.reshape(P2)
    tok = jnp.concatenate(
        [jnp.arange(N, dtype=jnp.int32), jnp.arange(N, dtype=jnp.int32)])
    gidx, wslot = _build_kernel()(slot, tok, wpair2d.reshape(P2))
    ysw = _ffn(be2d.reshape(3 * NB), gidx, x_flat, W1, W2,
               b1.reshape(E, 1, F), b2.reshape(E, 1, D), wslot.reshape(P, 1))
    out = _combine_kernel()(ysw, slot[:N], slot[N:])
    return out.reshape(x.shape), aux[0, 0]


# speedup vs baseline: 2.0227x; 1.0004x over previous
import functools

import jax
import jax.numpy as jnp
from jax import lax
from jax.experimental import pallas as pl
from jax.experimental.pallas import tpu as pltpu
import jax.experimental.pallas.tpu_sc as plsc

N = 2048
D = 768
E = 8
F = 3072
K = 2
P2 = 2 * N
BLK = 128
NB = P2 // BLK + E
P = NB * BLK

NC = 2
NS = 16
NW = NC * NS
LANES = 16

@functools.cache
def _sc_mesh():
    return plsc.VectorSubcoreMesh(
        core_axis_name="c", subcore_axis_name="s",
        num_cores=NC, num_subcores=NS)


def _router_body(rwT_ref, rb_ref, xT_ref, wpair_ref, slot_ref, be_ref, aux_ref):
    logits = jnp.dot(rwT_ref[...], xT_ref[...],
                     preferred_element_type=jnp.float32) + rb_ref[...]
    m = jnp.max(logits, axis=0, keepdims=True)
    ex = jnp.exp(logits - m)
    probs = ex / jnp.sum(ex, axis=0, keepdims=True)

    mp = jnp.sum(probs, axis=1, keepdims=True) * (1.0 / N)
    mu = jnp.sum(mp) * (1.0 / E)
    aux_ref[0, 0] = jnp.sum((mp - mu) ** 2) * (1.0 / (E - 1))

    eid = lax.broadcasted_iota(jnp.int32, (E, N), 0)
    p1 = jnp.max(probs, axis=0, keepdims=True)
    i1 = jnp.min(jnp.where(probs == p1, eid, E), axis=0, keepdims=True)
    oh1 = eid == i1
    masked = jnp.where(oh1, -1.0, probs)
    p2 = jnp.max(masked, axis=0, keepdims=True)
    i2 = jnp.min(jnp.where(masked == p2, eid, E), axis=0, keepdims=True)
    oh2 = eid == i2
    sw = p1 + p2
    wpair_ref[...] = jnp.concatenate([p1 / sw, p2 / sw], axis=1)

    oh = jnp.concatenate([oh1, oh2], axis=1).astype(jnp.float32)
    c = oh
    sh = 1
    while sh < P2:
        c = c + jnp.concatenate(
            [jnp.zeros((E, sh), jnp.float32), c[:, : P2 - sh]], axis=1)
        sh *= 2
    counts = c[:, P2 - 1 : P2]
    rank = c - oh
    caps = jnp.ceil(counts * (1.0 / BLK)) * BLK
    ic = caps
    sh = 1
    while sh < E:
        ic = ic + jnp.concatenate(
            [jnp.zeros((sh, 1), jnp.float32), ic[: E - sh]], axis=0)
        sh *= 2
    gs = ic - caps
    slot_f = jnp.sum(oh * (gs + rank), axis=0, keepdims=True)
    slot_ref[...] = slot_f.astype(jnp.int32)

    bstart = lax.broadcasted_iota(jnp.int32, (E, NB), 1) * BLK
    be = jnp.sum((bstart >= ic.astype(jnp.int32)).astype(jnp.int32),
                 axis=0, keepdims=True)
    be = jnp.minimum(be, E - 1)
    be_prev = jnp.concatenate([be[:, :1], be[:, : NB - 1]], axis=1)
    gord = (be != be_prev).astype(jnp.int32)
    sh = 1
    while sh < NB:
        gord = gord + jnp.concatenate(
            [jnp.zeros((1, sh), jnp.int32), gord[:, : NB - sh]], axis=1)
        sh *= 2
    eidN = lax.broadcasted_iota(jnp.int32, (E, NB), 0)
    present = caps > 0.0
    cand = jnp.where((eidN > be) & present, eidN, E)
    nxe = jnp.min(cand, axis=0, keepdims=True)
    nxe = jnp.where(nxe == E, be, nxe)
    be_ref[...] = jnp.concatenate([be, gord, nxe], axis=1)


_router = pl.pallas_call(
    _router_body,
    out_shape=(
        jax.ShapeDtypeStruct((1, P2), jnp.float32),
        jax.ShapeDtypeStruct((1, P2), jnp.int32),
        jax.ShapeDtypeStruct((1, 3 * NB), jnp.int32),
        jax.ShapeDtypeStruct((1, 1), jnp.float32),
    ),
    out_specs=(
        pl.BlockSpec(memory_space=pltpu.VMEM),
        pl.BlockSpec(memory_space=pltpu.VMEM),
        pl.BlockSpec(memory_space=pltpu.VMEM),
        pl.BlockSpec(memory_space=pltpu.SMEM),
    ),
)


@functools.cache
def _build_kernel():
    return pl.kernel(
        _build_body,
        out_type=(
            jax.ShapeDtypeStruct((P,), jnp.int32),
            jax.ShapeDtypeStruct((P,), jnp.float32),
        ),
        mesh=_sc_mesh(),
        compiler_params=pltpu.CompilerParams(needs_layout_passes=False),
        scratch_types=(
            pltpu.VMEM((P,), jnp.int32),
            pltpu.VMEM((P,), jnp.float32),
            pltpu.VMEM((P2,), jnp.int32),
            pltpu.VMEM((P2,), jnp.int32),
            pltpu.VMEM((P2,), jnp.float32),
        ),
    )


def _build_body(slot_hbm, tok_hbm, wp_hbm, gidx_hbm, wslot_hbm,
                g_v, w_v, slot_v, tok_v, wp_v):
    cid = lax.axis_index("c")
    sid = lax.axis_index("s")

    @pl.when(jnp.logical_and(cid == 0, sid == 0))
    def _():
        pltpu.sync_copy(slot_hbm, slot_v)
        pltpu.sync_copy(tok_hbm, tok_v)
        pltpu.sync_copy(wp_hbm, wp_v)

        def _zero(i, carry):
            g_v[pl.ds(i * LANES, LANES)] = jnp.zeros((LANES,), jnp.int32)
            w_v[pl.ds(i * LANES, LANES)] = jnp.zeros((LANES,), jnp.float32)
            return carry
        lax.fori_loop(0, P // LANES, _zero, 0)

        def _scat(i, carry):
            sl = slot_v[pl.ds(i * LANES, LANES)]
            plsc.store_scatter(g_v, [sl], tok_v[pl.ds(i * LANES, LANES)])
            plsc.store_scatter(w_v, [sl], wp_v[pl.ds(i * LANES, LANES)])
            return carry
        lax.fori_loop(0, P2 // LANES, _scat, 0)

        pltpu.sync_copy(g_v, gidx_hbm)
        pltpu.sync_copy(w_v, wslot_hbm)


def _gather_block(x_ref, gidx_ref, buf, sem, blk):
    for r in range(BLK):
        tid = gidx_ref[blk * BLK + r]
        pltpu.make_async_copy(
            x_ref.at[pl.ds(tid, 1)], buf.at[pl.ds(r, 1)], sem).start()


_NSLAB = 4
_DS = D // _NSLAB
_FS = F // _NSLAB


def _fire_weights(w1_any, w2_any, e, w1buf, w2buf, sem):
    for s in range(_NSLAB):
        pltpu.make_async_copy(
            w1_any.at[e, pl.ds(s * _DS, _DS)],
            w1buf.at[pl.ds(s * _DS, _DS)], sem).start()
        pltpu.make_async_copy(
            w2_any.at[e, pl.ds(s * _FS, _FS)],
            w2buf.at[pl.ds(s * _FS, _FS)], sem).start()


def _ffn_body(arr_ref, gidx_ref, x_ref, w1_any, w2_any, b1_ref, b2_ref,
              ws_ref, out_ref, xb0, xb1, w1a, w1b, w2a, w2b,
              sem0, sem1, wsa, wsb):
    i = pl.program_id(0)
    e = arr_ref[i]
    g = arr_ref[NB + i]
    nxe = arr_ref[2 * NB + i]
    g_prev = arr_ref[NB + jnp.maximum(i - 1, 0)]
    first = jnp.logical_or(i == 0, g != g_prev)
    podd = g % 2 == 1

    @pl.when(i == 0)
    def _():
        _gather_block(x_ref, gidx_ref, xb0, sem0, 0)
        _fire_weights(w1_any, w2_any, e, w1a, w2a, wsa)

    @pl.when(jnp.logical_and(i + 1 < NB, (i + 1) % 2 == 0))
    def _():
        _gather_block(x_ref, gidx_ref, xb0, sem0, i + 1)

    @pl.when(jnp.logical_and(i + 1 < NB, (i + 1) % 2 == 1))
    def _():
        _gather_block(x_ref, gidx_ref, xb1, sem1, i + 1)

    has_next = jnp.logical_and(first, nxe != e)

    @pl.when(jnp.logical_and(has_next, jnp.logical_not(podd)))
    def _():
        _fire_weights(w1_any, w2_any, nxe, w1b, w2b, wsb)

    @pl.when(jnp.logical_and(has_next, podd))
    def _():
        _fire_weights(w1_any, w2_any, nxe, w1a, w2a, wsa)

    @pl.when(jnp.logical_and(first, jnp.logical_not(podd)))
    def _():
        pltpu.make_async_copy(w1_any.at[e], w1a, wsa).wait()
        pltpu.make_async_copy(w2_any.at[e], w2a, wsa).wait()

    @pl.when(jnp.logical_and(first, podd))
    def _():
        pltpu.make_async_copy(w1_any.at[e], w1b, wsb).wait()
        pltpu.make_async_copy(w2_any.at[e], w2b, wsb).wait()

    def _compute(buf, sem, w1buf, w2buf):
        pltpu.make_async_copy(x_ref.at[pl.ds(0, BLK)], buf, sem).wait()
        h = jnp.dot(buf[...], w1buf[...],
                    preferred_element_type=jnp.float32) + b1_ref[0]
        h = 0.5 * h * (1.0 + lax.erf(h * 0.7071067811865476))
        y = jnp.dot(h, w2buf[...],
                    preferred_element_type=jnp.float32) + b2_ref[0]
        out_ref[...] = y * ws_ref[...]

    beven = i % 2 == 0

    @pl.when(jnp.logical_and(beven, jnp.logical_not(podd)))
    def _():
        _compute(xb0, sem0, w1a, w2a)

    @pl.when(jnp.logical_and(beven, podd))
    def _():
        _compute(xb0, sem0, w1b, w2b)

    @pl.when(jnp.logical_and(jnp.logical_not(beven), jnp.logical_not(podd)))
    def _():
        _compute(xb1, sem1, w1a, w2a)

    @pl.when(jnp.logical_and(jnp.logical_not(beven), podd))
    def _():
        _compute(xb1, sem1, w1b, w2b)


_ffn = pl.pallas_call(
    _ffn_body,
    grid_spec=pltpu.PrefetchScalarGridSpec(
        num_scalar_prefetch=2,
        grid=(NB,),
        in_specs=[
            pl.BlockSpec(memory_space=pl.ANY),
            pl.BlockSpec(memory_space=pl.ANY),
            pl.BlockSpec(memory_space=pl.ANY),
            pl.BlockSpec((1, 1, F), lambda i, arr, gidx: (arr[i], 0, 0)),
            pl.BlockSpec((1, 1, D), lambda i, arr, gidx: (arr[i], 0, 0)),
            pl.BlockSpec((BLK, 1), lambda i, arr, gidx: (i, 0)),
        ],
        out_specs=pl.BlockSpec((BLK, D), lambda i, arr, gidx: (i, 0)),
        scratch_shapes=[
            pltpu.VMEM((BLK, D), jnp.float32),
            pltpu.VMEM((BLK, D), jnp.float32),
            pltpu.VMEM((D, F), jnp.float32),
            pltpu.VMEM((D, F), jnp.float32),
            pltpu.VMEM((F, D), jnp.float32),
            pltpu.VMEM((F, D), jnp.float32),
            pltpu.SemaphoreType.DMA,
            pltpu.SemaphoreType.DMA,
            pltpu.SemaphoreType.DMA,
            pltpu.SemaphoreType.DMA,
        ],
    ),
    out_shape=jax.ShapeDtypeStruct((P, D), jnp.float32),
    compiler_params=pltpu.CompilerParams(
        dimension_semantics=("arbitrary",)),
)


_TOK_W = N // NW


@functools.cache
def _combine_kernel():
    return pl.kernel(
        _combine_body,
        out_type=jax.ShapeDtypeStruct((N, D), jnp.float32),
        mesh=_sc_mesh(),
        compiler_params=pltpu.CompilerParams(needs_layout_passes=False),
        scratch_types=(
            pltpu.VMEM((_TOK_W,), jnp.int32),
            pltpu.VMEM((_TOK_W,), jnp.int32),
            pltpu.VMEM((_TOK_W, D), jnp.float32),
            pltpu.VMEM((_TOK_W, D), jnp.float32),
            pltpu.SemaphoreType.DMA,
            pltpu.SemaphoreType.DMA,
        ),
    )


def _combine_body(ysw_hbm, s1_hbm, s2_hbm, out_hbm,
                  idx1_v, idx2_v, buf1_v, buf2_v, sem1, sem2):
    cid = lax.axis_index("c")
    sid = lax.axis_index("s")
    wid = sid * NC + cid
    base = wid * _TOK_W
    pltpu.sync_copy(s1_hbm.at[pl.ds(base, _TOK_W)], idx1_v)
    pltpu.sync_copy(s2_hbm.at[pl.ds(base, _TOK_W)], idx2_v)
    cp1 = pltpu.async_copy(ysw_hbm.at[idx1_v], buf1_v, sem1)
    cp2 = pltpu.async_copy(ysw_hbm.at[idx2_v], buf2_v, sem2)
    cp1.wait()
    cp2.wait()

    def _row(r, carry):
        for j in range(D // LANES):
            sl = pl.ds(j * LANES, LANES)
            buf1_v[r, sl] = buf1_v[r, sl] + buf2_v[r, sl]
        return carry
    lax.fori_loop(0, _TOK_W, _row, 0)
    pltpu.sync_copy(buf1_v, out_hbm.at[pl.ds(base, _TOK_W)])


def kernel(x, router_W, router_b, W1, b1, W2, b2):
    x_flat = x.reshape(N, D)
    wpair2d, slot2d, be2d, aux = _router(
        router_W.T, router_b.reshape(E, 1), x_flat.T)
    slot = slot2d.reshape(P2)
    tok = jnp.concatenate(
        [jnp.arange(N, dtype=jnp.int32), jnp.arange(N, dtype=jnp.int32)])
    gidx, wslot = _build_kernel()(slot, tok, wpair2d.reshape(P2))
    ysw = _ffn(be2d.reshape(3 * NB), gidx, x_flat, W1, W2,
               b1.reshape(E, 1, F), b2.reshape(E, 1, D), wslot.reshape(P, 1))
    out = _combine_kernel()(ysw, slot[:N], slot[N:])
    return out.reshape(x.shape), aux[0, 0]


# speedup vs baseline: 2.1721x; 1.0738x over previous
import functools

import jax
import jax.numpy as jnp
from jax import lax
from jax.experimental import pallas as pl
from jax.experimental.pallas import tpu as pltpu
import jax.experimental.pallas.tpu_sc as plsc

N = 2048
D = 768
E = 8
F = 3072
K = 2
P2 = 2 * N
BLK = 128
NB = P2 // BLK + E
P = NB * BLK

NC = 2
NS = 16
NW = NC * NS
LANES = 16

@functools.cache
def _sc_mesh():
    return plsc.VectorSubcoreMesh(
        core_axis_name="c", subcore_axis_name="s",
        num_cores=NC, num_subcores=NS)


def _router_body(rwT_ref, rb_ref, x_ref, wpair_ref, slot1_ref, slot2_ref,
                 be_ref, aux_ref):
    logits = lax.dot_general(
        rwT_ref[...], x_ref[...], (((1,), (1,)), ((), ())),
        preferred_element_type=jnp.float32) + rb_ref[...]
    m = jnp.max(logits, axis=0, keepdims=True)
    ex = jnp.exp(logits - m)
    probs = ex / jnp.sum(ex, axis=0, keepdims=True)

    mp = jnp.sum(probs, axis=1, keepdims=True) * (1.0 / N)
    mu = jnp.sum(mp) * (1.0 / E)
    aux_ref[0, 0] = jnp.sum((mp - mu) ** 2) * (1.0 / (E - 1))

    eid = lax.broadcasted_iota(jnp.int32, (E, N), 0)
    p1 = jnp.max(probs, axis=0, keepdims=True)
    i1 = jnp.min(jnp.where(probs == p1, eid, E), axis=0, keepdims=True)
    oh1 = eid == i1
    masked = jnp.where(oh1, -1.0, probs)
    p2 = jnp.max(masked, axis=0, keepdims=True)
    i2 = jnp.min(jnp.where(masked == p2, eid, E), axis=0, keepdims=True)
    oh2 = eid == i2
    sw = p1 + p2
    wpair_ref[...] = jnp.concatenate([p1 / sw, p2 / sw], axis=1)

    oh = jnp.concatenate([oh1, oh2], axis=1).astype(jnp.float32)
    c = oh
    sh = 1
    while sh < P2:
        c = c + jnp.concatenate(
            [jnp.zeros((E, sh), jnp.float32), c[:, : P2 - sh]], axis=1)
        sh *= 2
    counts = c[:, P2 - 1 : P2]
    rank = c - oh
    caps = jnp.ceil(counts * (1.0 / BLK)) * BLK
    ic = caps
    sh = 1
    while sh < E:
        ic = ic + jnp.concatenate(
            [jnp.zeros((sh, 1), jnp.float32), ic[: E - sh]], axis=0)
        sh *= 2
    gs = ic - caps
    slot_f = jnp.sum(oh * (gs + rank), axis=0, keepdims=True)
    slot_i = slot_f.astype(jnp.int32)
    slot1_ref[...] = slot_i[:, :N]
    slot2_ref[...] = slot_i[:, N:]

    bstart = lax.broadcasted_iota(jnp.int32, (E, NB), 1) * BLK
    be = jnp.sum((bstart >= ic.astype(jnp.int32)).astype(jnp.int32),
                 axis=0, keepdims=True)
    be = jnp.minimum(be, E - 1)
    be_prev = jnp.concatenate([be[:, :1], be[:, : NB - 1]], axis=1)
    gord = (be != be_prev).astype(jnp.int32)
    sh = 1
    while sh < NB:
        gord = gord + jnp.concatenate(
            [jnp.zeros((1, sh), jnp.int32), gord[:, : NB - sh]], axis=1)
        sh *= 2
    eidN = lax.broadcasted_iota(jnp.int32, (E, NB), 0)
    present = caps > 0.0
    cand = jnp.where((eidN > be) & present, eidN, E)
    nxe = jnp.min(cand, axis=0, keepdims=True)
    nxe = jnp.where(nxe == E, be, nxe)
    be_ref[...] = jnp.concatenate([be, gord, nxe], axis=1)


_router = pl.pallas_call(
    _router_body,
    out_shape=(
        jax.ShapeDtypeStruct((1, P2), jnp.float32),
        jax.ShapeDtypeStruct((1, N), jnp.int32),
        jax.ShapeDtypeStruct((1, N), jnp.int32),
        jax.ShapeDtypeStruct((1, 3 * NB), jnp.int32),
        jax.ShapeDtypeStruct((1, 1), jnp.float32),
    ),
    out_specs=(
        pl.BlockSpec(memory_space=pltpu.VMEM),
        pl.BlockSpec(memory_space=pltpu.VMEM),
        pl.BlockSpec(memory_space=pltpu.VMEM),
        pl.BlockSpec(memory_space=pltpu.VMEM),
        pl.BlockSpec(memory_space=pltpu.SMEM),
    ),
)


@functools.cache
def _build_kernel():
    return pl.kernel(
        _build_body,
        out_type=(
            jax.ShapeDtypeStruct((P,), jnp.int32),
            jax.ShapeDtypeStruct((P,), jnp.float32),
        ),
        mesh=_sc_mesh(),
        compiler_params=pltpu.CompilerParams(needs_layout_passes=False),
        scratch_types=(
            pltpu.VMEM((P,), jnp.int32),
            pltpu.VMEM((P,), jnp.float32),
            pltpu.VMEM((N,), jnp.int32),
            pltpu.VMEM((N,), jnp.int32),
            pltpu.VMEM((P2,), jnp.float32),
        ),
    )


def _build_body(slot1_hbm, slot2_hbm, wp_hbm, gidx_hbm, wslot_hbm,
                g_v, w_v, s1_v, s2_v, wp_v):
    cid = lax.axis_index("c")
    sid = lax.axis_index("s")

    @pl.when(jnp.logical_and(cid == 0, sid == 0))
    def _():
        pltpu.sync_copy(slot1_hbm, s1_v)
        pltpu.sync_copy(slot2_hbm, s2_v)
        pltpu.sync_copy(wp_hbm, wp_v)

        def _zero(i, carry):
            g_v[pl.ds(i * LANES, LANES)] = jnp.zeros((LANES,), jnp.int32)
            w_v[pl.ds(i * LANES, LANES)] = jnp.zeros((LANES,), jnp.float32)
            return carry
        lax.fori_loop(0, P // LANES, _zero, 0)

        tok0 = lax.iota(jnp.int32, LANES)

        def _scat(i, carry):
            tok = tok0 + i * LANES
            sl1 = s1_v[pl.ds(i * LANES, LANES)]
            plsc.store_scatter(g_v, [sl1], tok)
            plsc.store_scatter(w_v, [sl1], wp_v[pl.ds(i * LANES, LANES)])
            sl2 = s2_v[pl.ds(i * LANES, LANES)]
            plsc.store_scatter(g_v, [sl2], tok)
            plsc.store_scatter(
                w_v, [sl2], wp_v[pl.ds(N + i * LANES, LANES)])
            return carry
        lax.fori_loop(0, N // LANES, _scat, 0)

        pltpu.sync_copy(g_v, gidx_hbm)
        pltpu.sync_copy(w_v, wslot_hbm)


def _gather_block(x_ref, gidx_ref, buf, sem, blk):
    for r in range(BLK):
        tid = gidx_ref[blk * BLK + r]
        pltpu.make_async_copy(
            x_ref.at[pl.ds(tid, 1)], buf.at[pl.ds(r, 1)], sem).start()


_NSLAB = 4
_DS = D // _NSLAB
_FS = F // _NSLAB


def _fire_weights(w1_any, w2_any, e, w1buf, w2buf, sem):
    for s in range(_NSLAB):
        pltpu.make_async_copy(
            w1_any.at[e, pl.ds(s * _DS, _DS)],
            w1buf.at[pl.ds(s * _DS, _DS)], sem).start()
        pltpu.make_async_copy(
            w2_any.at[e, pl.ds(s * _FS, _FS)],
            w2buf.at[pl.ds(s * _FS, _FS)], sem).start()


def _ffn_body(arr_ref, gidx_ref, x_ref, w1_any, w2_any, b1_ref, b2_ref,
              ws_ref, out_ref, xb0, xb1, w1a, w1b, w2a, w2b,
              sem0, sem1, wsa, wsb):
    i = pl.program_id(0)
    e = arr_ref[i]
    g = arr_ref[NB + i]
    nxe = arr_ref[2 * NB + i]
    g_prev = arr_ref[NB + jnp.maximum(i - 1, 0)]
    first = jnp.logical_or(i == 0, g != g_prev)
    podd = g % 2 == 1

    @pl.when(i == 0)
    def _():
        _gather_block(x_ref, gidx_ref, xb0, sem0, 0)
        _fire_weights(w1_any, w2_any, e, w1a, w2a, wsa)

    @pl.when(jnp.logical_and(i + 1 < NB, (i + 1) % 2 == 0))
    def _():
        _gather_block(x_ref, gidx_ref, xb0, sem0, i + 1)

    @pl.when(jnp.logical_and(i + 1 < NB, (i + 1) % 2 == 1))
    def _():
        _gather_block(x_ref, gidx_ref, xb1, sem1, i + 1)

    has_next = jnp.logical_and(first, nxe != e)

    @pl.when(jnp.logical_and(has_next, jnp.logical_not(podd)))
    def _():
        _fire_weights(w1_any, w2_any, nxe, w1b, w2b, wsb)

    @pl.when(jnp.logical_and(has_next, podd))
    def _():
        _fire_weights(w1_any, w2_any, nxe, w1a, w2a, wsa)

    @pl.when(jnp.logical_and(first, jnp.logical_not(podd)))
    def _():
        pltpu.make_async_copy(w1_any.at[e], w1a, wsa).wait()
        pltpu.make_async_copy(w2_any.at[e], w2a, wsa).wait()

    @pl.when(jnp.logical_and(first, podd))
    def _():
        pltpu.make_async_copy(w1_any.at[e], w1b, wsb).wait()
        pltpu.make_async_copy(w2_any.at[e], w2b, wsb).wait()

    def _compute(buf, sem, w1buf, w2buf):
        pltpu.make_async_copy(x_ref.at[pl.ds(0, BLK)], buf, sem).wait()
        h = jnp.dot(buf[...], w1buf[...],
                    preferred_element_type=jnp.float32) + b1_ref[0]
        h = 0.5 * h * (1.0 + lax.erf(h * 0.7071067811865476))
        y = jnp.dot(h, w2buf[...],
                    preferred_element_type=jnp.float32) + b2_ref[0]
        out_ref[...] = y * ws_ref[...]

    beven = i % 2 == 0

    @pl.when(jnp.logical_and(beven, jnp.logical_not(podd)))
    def _():
        _compute(xb0, sem0, w1a, w2a)

    @pl.when(jnp.logical_and(beven, podd))
    def _():
        _compute(xb0, sem0, w1b, w2b)

    @pl.when(jnp.logical_and(jnp.logical_not(beven), jnp.logical_not(podd)))
    def _():
        _compute(xb1, sem1, w1a, w2a)

    @pl.when(jnp.logical_and(jnp.logical_not(beven), podd))
    def _():
        _compute(xb1, sem1, w1b, w2b)


_ffn = pl.pallas_call(
    _ffn_body,
    grid_spec=pltpu.PrefetchScalarGridSpec(
        num_scalar_prefetch=2,
        grid=(NB,),
        in_specs=[
            pl.BlockSpec(memory_space=pl.ANY),
            pl.BlockSpec(memory_space=pl.ANY),
            pl.BlockSpec(memory_space=pl.ANY),
            pl.BlockSpec((1, 1, F), lambda i, arr, gidx: (arr[i], 0, 0)),
            pl.BlockSpec((1, 1, D), lambda i, arr, gidx: (arr[i], 0, 0)),
            pl.BlockSpec((BLK, 1), lambda i, arr, gidx: (i, 0)),
        ],
        out_specs=pl.BlockSpec((BLK, D), lambda i, arr, gidx: (i, 0)),
        scratch_shapes=[
            pltpu.VMEM((BLK, D), jnp.float32),
            pltpu.VMEM((BLK, D), jnp.float32),
            pltpu.VMEM((D, F), jnp.float32),
            pltpu.VMEM((D, F), jnp.float32),
            pltpu.VMEM((F, D), jnp.float32),
            pltpu.VMEM((F, D), jnp.float32),
            pltpu.SemaphoreType.DMA,
            pltpu.SemaphoreType.DMA,
            pltpu.SemaphoreType.DMA,
            pltpu.SemaphoreType.DMA,
        ],
    ),
    out_shape=jax.ShapeDtypeStruct((P, D), jnp.float32),
    compiler_params=pltpu.CompilerParams(
        dimension_semantics=("arbitrary",)),
)


_TOK_W = N // NW


@functools.cache
def _combine_kernel():
    return pl.kernel(
        _combine_body,
        out_type=jax.ShapeDtypeStruct((N, D), jnp.float32),
        mesh=_sc_mesh(),
        compiler_params=pltpu.CompilerParams(needs_layout_passes=False),
        scratch_types=(
            pltpu.VMEM((_TOK_W,), jnp.int32),
            pltpu.VMEM((_TOK_W,), jnp.int32),
            pltpu.VMEM((_TOK_W, D), jnp.float32),
            pltpu.VMEM((_TOK_W, D), jnp.float32),
            pltpu.SemaphoreType.DMA,
            pltpu.SemaphoreType.DMA,
        ),
    )


def _combine_body(ysw_hbm, s1_hbm, s2_hbm, out_hbm,
                  idx1_v, idx2_v, buf1_v, buf2_v, sem1, sem2):
    cid = lax.axis_index("c")
    sid = lax.axis_index("s")
    wid = sid * NC + cid
    base = wid * _TOK_W
    pltpu.sync_copy(s1_hbm.at[pl.ds(base, _TOK_W)], idx1_v)
    pltpu.sync_copy(s2_hbm.at[pl.ds(base, _TOK_W)], idx2_v)
    cp1 = pltpu.async_copy(ysw_hbm.at[idx1_v], buf1_v, sem1)
    cp2 = pltpu.async_copy(ysw_hbm.at[idx2_v], buf2_v, sem2)
    cp1.wait()
    cp2.wait()

    def _row(r, carry):
        for j in range(D // LANES):
            sl = pl.ds(j * LANES, LANES)
            buf1_v[r, sl] = buf1_v[r, sl] + buf2_v[r, sl]
        return carry
    lax.fori_loop(0, _TOK_W, _row, 0)
    pltpu.sync_copy(buf1_v, out_hbm.at[pl.ds(base, _TOK_W)])


def kernel(x, router_W, router_b, W1, b1, W2, b2):
    x_flat = x.reshape(N, D)
    wpair2d, slot1, slot2, be2d, aux = _router(
        router_W.T, router_b.reshape(E, 1), x_flat)
    gidx, wslot = _build_kernel()(
        slot1.reshape(N), slot2.reshape(N), wpair2d.reshape(P2))
    ysw = _ffn(be2d.reshape(3 * NB), gidx, x_flat, W1, W2,
               b1.reshape(E, 1, F), b2.reshape(E, 1, D), wslot.reshape(P, 1))
    out = _combine_kernel()(ysw, slot1.reshape(N), slot2.reshape(N))
    return out.reshape(x.shape), aux[0, 0]
